# baseline (speedup 1.0000x reference)
import jax
import jax.numpy as jnp
from jax.experimental import pallas as pl
from jax.experimental.pallas import tpu as pltpu

B, N, E = 16, 256, 512
NODE_IN, EDGE_IN, NODE_OUT, EDGE_OUT = 128, 16, 128, 16
BB = 2

_F32 = jnp.float32
_BF16 = jnp.bfloat16


def _dot(a, b):
    return jnp.dot(a, b, preferred_element_type=_F32)


def _dg(a, b, dims):
    return jax.lax.dot_general(a, b, (dims, ((), ())),
                               preferred_element_type=_F32)


def _bf(x):
    return x.astype(_BF16)


def _kernel(node_ref, edge_ref, node_adj_ref, edge_adj_ref, D_v_ref, D_e_ref,
            T_ref, Wn1_ref, We1_ref, pe1_ref, pv1_ref, Wn2_ref, We2_ref,
            pe2_ref, pv2_ref, Wv1_ref, bv1_ref, Wv2_ref, bv2_ref, out_ref):
    R = range(BB)
    n = [node_ref[i] for i in R]
    e = [edge_ref[i] for i in R]
    Av = [node_adj_ref[i] for i in R]
    Ae = [edge_adj_ref[i] for i in R]
    Dv = [D_v_ref[i] for i in R]
    De = [D_e_ref[i] for i in R]
    Tm = [T_ref[i] for i in R]

    Tb = [_bf(Tm[i]) for i in R]
    Dvb = [_bf(Dv[i]) for i in R]
    Deb = [_bf(De[i]) for i in R]

    def node_prop(n, deT, Wn):
        Tde = [_bf(Tm[i] * deT[i]) for i in R]
        A_node = [_dg(Tde[i], Tb[i], ((1,), (1,))) * Av[i] for i in R]
        x = [_dot(n[i], Wn) for i in R]
        x = [_dot(Dvb[i], _bf(x[i])) for i in R]
        x = [_dot(_bf(A_node[i]), _bf(x[i])) for i in R]
        return [jax.nn.relu(_dot(Dvb[i], _bf(x[i]))) for i in R]

    de1T = [_dg(pe1_ref[...], e[i], ((0,), (1,))) for i in R]
    n1 = node_prop(n, de1T, Wn1_ref[...])

    dv1 = [_dot(n[i], pv1_ref[...]) for i in R]
    Tdv = [_bf(Tm[i] * dv1[i]) for i in R]
    A_edge = [_dg(Tdv[i], Tb[i], ((0,), (0,))) * Ae[i] for i in R]
    yT = [_dg(We1_ref[...], e[i], ((0,), (1,))) for i in R]
    yT = [_dg(_bf(yT[i]), Deb[i], ((1,), (1,))) for i in R]
    yT = [_dg(_bf(yT[i]), _bf(A_edge[i]), ((1,), (1,))) for i in R]
    e1T = [jax.nn.relu(_dg(_bf(yT[i]), Deb[i], ((1,), (1,)))) for i in R]

    de2T = [_dg(pe2_ref[...], e1T[i], ((0,), (0,))) for i in R]
    n2 = node_prop(n1, de2T, Wn2_ref[...])

    v = [jax.nn.relu(_dot(n2[i], Wv1_ref[...]) + bv1_ref[...][None, :]) for i in R]
    vm = [jnp.mean(v[i], axis=0, keepdims=True) for i in R]
    for i in R:
        out_ref[i] = _dot(vm[i], Wv2_ref[...]) + bv2_ref[...][None, :]


def kernel(node, edge, node_adj, edge_adj, D_v, D_e, T,
           Wn1, We1, pe1, pv1, Wn2, We2, pe2, pv2,
           Wv1, bv1, Wv2, bv2):
    batch = lambda *dims: pl.BlockSpec((BB,) + dims, lambda b: (b, 0, 0))
    full = lambda arr: pl.BlockSpec(arr.shape, lambda b: (0,) * arr.ndim)
    grid_spec = pl.GridSpec(
        grid=(B // BB,),
        in_specs=[
            batch(N, NODE_IN),
            batch(E, EDGE_IN),
            batch(N, N),
            batch(E, E),
            batch(N, N),
            batch(E, E),
            batch(N, E),
            full(Wn1), full(We1), full(pe1), full(pv1),
            full(Wn2), full(We2), full(pe2), full(pv2),
            full(Wv1), full(bv1), full(Wv2), full(bv2),
        ],
        out_specs=pl.BlockSpec((BB, 1, 1), lambda b: (b, 0, 0)),
    )
    out = pl.pallas_call(
        _kernel,
        grid_spec=grid_spec,
        out_shape=jax.ShapeDtypeStruct((B, 1, 1), jnp.float32),
        compiler_params=pltpu.CompilerParams(
            dimension_semantics=("parallel",),
        ),
    )(node, edge, node_adj, edge_adj, D_v, D_e, T,
      Wn1, We1, pe1, pv1, Wn2, We2, pe2, pv2,
      Wv1, bv1, Wv2, bv2)
    return out.reshape(B, 1)


# speedup vs baseline: 1.1585x; 1.1585x over previous
import jax
import jax.numpy as jnp
from jax.experimental import pallas as pl
from jax.experimental.pallas import tpu as pltpu

B, N, E = 16, 256, 512
NODE_IN, EDGE_IN, NODE_OUT, EDGE_OUT = 128, 16, 128, 16
BB = 4

_F32 = jnp.float32
_BF16 = jnp.bfloat16


def _dot(a, b):
    return jnp.dot(a, b, preferred_element_type=_F32)


def _dg(a, b, dims):
    return jax.lax.dot_general(a, b, (dims, ((), ())),
                               preferred_element_type=_F32)


def _bf(x):
    return x.astype(_BF16)


def _kernel(node_ref, edge_ref, node_adj_ref, edge_adj_ref, D_v_ref, D_e_ref,
            T_ref, Wn1_ref, We1_ref, pe1_ref, pv1_ref, Wn2_ref, We2_ref,
            pe2_ref, pv2_ref, Wv1_ref, bv1_ref, Wv2_ref, bv2_ref, out_ref):
    R = range(BB)
    n = [node_ref[i] for i in R]
    e = [edge_ref[i] for i in R]
    Av = [node_adj_ref[i] for i in R]
    Ae = [edge_adj_ref[i] for i in R]
    Dv = [D_v_ref[i] for i in R]
    De = [D_e_ref[i] for i in R]
    Tm = [T_ref[i] for i in R]

    Tb = [_bf(Tm[i]) for i in R]
    Dvb = [_bf(Dv[i]) for i in R]
    Deb = [_bf(De[i]) for i in R]

    def node_prop(n, deT, Wn):
        Tde = [_bf(Tm[i] * deT[i]) for i in R]
        A_node = [_dg(Tde[i], Tb[i], ((1,), (1,))) * Av[i] for i in R]
        x = [_dot(n[i], Wn) for i in R]
        x = [_dot(Dvb[i], _bf(x[i])) for i in R]
        x = [_dot(_bf(A_node[i]), _bf(x[i])) for i in R]
        return [jax.nn.relu(_dot(Dvb[i], _bf(x[i]))) for i in R]

    de1T = [_dg(pe1_ref[...], e[i], ((0,), (1,))) for i in R]
    n1 = node_prop(n, de1T, Wn1_ref[...])

    dv1 = [_dot(n[i], pv1_ref[...]) for i in R]
    Tdv = [_bf(Tm[i] * dv1[i]) for i in R]
    A_edge = [_dg(Tdv[i], Tb[i], ((0,), (0,))) * Ae[i] for i in R]
    yT = [_dg(We1_ref[...], e[i], ((0,), (1,))) for i in R]
    yT = [_dg(_bf(yT[i]), Deb[i], ((1,), (1,))) for i in R]
    yT = [_dg(_bf(yT[i]), _bf(A_edge[i]), ((1,), (1,))) for i in R]
    e1T = [jax.nn.relu(_dg(_bf(yT[i]), Deb[i], ((1,), (1,)))) for i in R]

    de2T = [_dg(pe2_ref[...], e1T[i], ((0,), (0,))) for i in R]
    n2 = node_prop(n1, de2T, Wn2_ref[...])

    v = [jax.nn.relu(_dot(n2[i], Wv1_ref[...]) + bv1_ref[...][None, :]) for i in R]
    vm = [jnp.mean(v[i], axis=0, keepdims=True) for i in R]
    for i in R:
        out_ref[i] = _dot(vm[i], Wv2_ref[...]) + bv2_ref[...][None, :]


def kernel(node, edge, node_adj, edge_adj, D_v, D_e, T,
           Wn1, We1, pe1, pv1, Wn2, We2, pe2, pv2,
           Wv1, bv1, Wv2, bv2):
    batch = lambda *dims: pl.BlockSpec((BB,) + dims, lambda b: (b, 0, 0))
    full = lambda arr: pl.BlockSpec(arr.shape, lambda b: (0,) * arr.ndim)
    grid_spec = pl.GridSpec(
        grid=(B // BB,),
        in_specs=[
            batch(N, NODE_IN),
            batch(E, EDGE_IN),
            batch(N, N),
            batch(E, E),
            batch(N, N),
            batch(E, E),
            batch(N, E),
            full(Wn1), full(We1), full(pe1), full(pv1),
            full(Wn2), full(We2), full(pe2), full(pv2),
            full(Wv1), full(bv1), full(Wv2), full(bv2),
        ],
        out_specs=pl.BlockSpec((BB, 1, 1), lambda b: (b, 0, 0)),
    )
    out = pl.pallas_call(
        _kernel,
        grid_spec=grid_spec,
        out_shape=jax.ShapeDtypeStruct((B, 1, 1), jnp.float32),
        compiler_params=pltpu.CompilerParams(
            dimension_semantics=("arbitrary",),
        ),
    )(node, edge, node_adj, edge_adj, D_v, D_e, T,
      Wn1, We1, pe1, pv1, Wn2, We2, pe2, pv2,
      Wv1, bv1, Wv2, bv2)
    return out.reshape(B, 1)


# speedup vs baseline: 1.6541x; 1.4279x over previous
import jax
import jax.numpy as jnp
from jax.experimental import pallas as pl
from jax.experimental.pallas import tpu as pltpu

B, N, E = 16, 256, 512
NODE_IN, EDGE_IN, NODE_OUT, EDGE_OUT = 128, 16, 128, 16
BB = 4

_F32 = jnp.float32
_BF16 = jnp.bfloat16


def _dot(a, b):
    return jnp.dot(a, b, preferred_element_type=_F32)


def _dg(a, b, dims):
    return jax.lax.dot_general(a, b, (dims, ((), ())),
                               preferred_element_type=_F32)


def _bf(x):
    return x.astype(_BF16)


def _kernel(node_ref, edgeT_ref, node_adj_ref, edge_adj_ref, D_v_ref, D_e_ref,
            T_ref, Wn1_ref, We1_ref, pe1_ref, pv1_ref, Wn2_ref,
            pe2_ref, pv2_ref, Wv1_ref, bv1_ref, Wv2_ref, bv2_ref, out_ref):
    R = range(BB)
    n = [node_ref[i] for i in R]
    eT = [edgeT_ref[i] for i in R]
    Av = [node_adj_ref[i] for i in R]
    Ae = [edge_adj_ref[i] for i in R]
    Dv = [D_v_ref[i] for i in R]
    De = [D_e_ref[i] for i in R]
    Tm = [T_ref[i] for i in R]

    Tb = [_bf(Tm[i]) for i in R]
    Dvb = [_bf(Dv[i]) for i in R]
    Deb = [_bf(De[i]) for i in R]

    def node_prop(n, deT, Wn):
        Tde = [_bf(Tm[i] * deT[i]) for i in R]
        A_node = [_dg(Tde[i], Tb[i], ((1,), (1,))) * Av[i] for i in R]
        x = [_dot(n[i], Wn) for i in R]
        x = [_dot(Dvb[i], _bf(x[i])) for i in R]
        x = [_dot(_bf(A_node[i]), _bf(x[i])) for i in R]
        return [jax.nn.relu(_dot(Dvb[i], _bf(x[i]))) for i in R]

    de1T = [_dg(pe1_ref[...], eT[i], ((1,), (0,))) for i in R]
    n1 = node_prop(n, de1T, Wn1_ref[...])

    dv1 = [_dg(n[i], pv1_ref[...], ((1,), (1,))) for i in R]
    Tdv = [_bf(Tm[i] * dv1[i]) for i in R]
    A_edge = [_dg(Tdv[i], Tb[i], ((0,), (0,))) * Ae[i] for i in R]
    yT = [_dg(We1_ref[...], eT[i], ((0,), (0,))) for i in R]
    yT = [_dg(_bf(yT[i]), Deb[i], ((1,), (1,))) for i in R]
    yT = [_dg(_bf(yT[i]), _bf(A_edge[i]), ((1,), (1,))) for i in R]
    e1T = [jax.nn.relu(_dg(_bf(yT[i]), Deb[i], ((1,), (1,)))) for i in R]

    de2T = [_dg(pe2_ref[...], e1T[i], ((1,), (0,))) for i in R]
    n2 = node_prop(n1, de2T, Wn2_ref[...])

    v = [jax.nn.relu(_dot(n2[i], Wv1_ref[...]) + bv1_ref[...][None, :]) for i in R]
    vm = [jnp.mean(v[i], axis=0, keepdims=True) for i in R]
    for i in R:
        out_ref[i] = (jnp.sum(vm[i] * Wv2_ref[...], axis=1, keepdims=True)
                      + bv2_ref[...][None, :])


def kernel(node, edge, node_adj, edge_adj, D_v, D_e, T,
           Wn1, We1, pe1, pv1, Wn2, We2, pe2, pv2,
           Wv1, bv1, Wv2, bv2):
    edgeT = jnp.swapaxes(edge, 1, 2)
    pe1r = pe1.reshape(1, EDGE_IN)
    pv1r = pv1.reshape(1, NODE_IN)
    pe2r = pe2.reshape(1, EDGE_OUT)
    pv2r = pv2.reshape(1, NODE_OUT)
    Wv2r = Wv2.reshape(1, NODE_OUT)
    batch = lambda *dims: pl.BlockSpec((BB,) + dims, lambda b: (b, 0, 0))
    full = lambda arr: pl.BlockSpec(arr.shape, lambda b: (0,) * arr.ndim)
    grid_spec = pl.GridSpec(
        grid=(B // BB,),
        in_specs=[
            batch(N, NODE_IN),
            batch(EDGE_IN, E),
            batch(N, N),
            batch(E, E),
            batch(N, N),
            batch(E, E),
            batch(N, E),
            full(Wn1), full(We1), full(pe1r), full(pv1r),
            full(Wn2), full(pe2r), full(pv2r),
            full(Wv1), full(bv1), full(Wv2r), full(bv2),
        ],
        out_specs=pl.BlockSpec((BB, 1, 1), lambda b: (b, 0, 0)),
    )
    out = pl.pallas_call(
        _kernel,
        grid_spec=grid_spec,
        out_shape=jax.ShapeDtypeStruct((B, 1, 1), jnp.float32),
        compiler_params=pltpu.CompilerParams(
            dimension_semantics=("arbitrary",),
        ),
    )(node, edgeT, node_adj, edge_adj, D_v, D_e, T,
      Wn1, We1, pe1r, pv1r, Wn2, pe2r, pv2r,
      Wv1, bv1, Wv2r, bv2)
    return out.reshape(B, 1)


# speedup vs baseline: 1.6582x; 1.0025x over previous
import jax
import jax.numpy as jnp
from jax.experimental import pallas as pl
from jax.experimental.pallas import tpu as pltpu

B, N, E = 16, 256, 512
NODE_IN, EDGE_IN, NODE_OUT, EDGE_OUT = 128, 16, 128, 16
BB = 4

_F32 = jnp.float32
_BF16 = jnp.bfloat16


def _dot(a, b):
    return jnp.dot(a, b, preferred_element_type=_F32)


def _dg(a, b, dims):
    return jax.lax.dot_general(a, b, (dims, ((), ())),
                               preferred_element_type=_F32)


def _bf(x):
    return x


def _kernel(node_ref, edgeT_ref, node_adj_ref, edge_adj_ref, D_v_ref, D_e_ref,
            T_ref, Wn1_ref, We1_ref, pe1_ref, pv1_ref, Wn2_ref,
            pe2_ref, pv2_ref, Wv1_ref, bv1_ref, Wv2_ref, bv2_ref, out_ref):
    R = range(BB)
    n = [node_ref[i] for i in R]
    eT = [edgeT_ref[i] for i in R]
    Av = [node_adj_ref[i] for i in R]
    Ae = [edge_adj_ref[i] for i in R]
    Dv = [D_v_ref[i] for i in R]
    De = [D_e_ref[i] for i in R]
    Tm = [T_ref[i] for i in R]

    Tb = [_bf(Tm[i]) for i in R]
    Dvb = [_bf(Dv[i]) for i in R]
    Deb = [_bf(De[i]) for i in R]

    def node_prop(n, deT, Wn):
        Tde = [_bf(Tm[i] * deT[i]) for i in R]
        A_node = [_dg(Tde[i], Tb[i], ((1,), (1,))) * Av[i] for i in R]
        x = [_dot(n[i], Wn) for i in R]
        x = [_dot(Dvb[i], _bf(x[i])) for i in R]
        x = [_dot(_bf(A_node[i]), _bf(x[i])) for i in R]
        return [jax.nn.relu(_dot(Dvb[i], _bf(x[i]))) for i in R]

    de1T = [_dg(pe1_ref[...], eT[i], ((1,), (0,))) for i in R]
    n1 = node_prop(n, de1T, Wn1_ref[...])

    dv1 = [_dg(n[i], pv1_ref[...], ((1,), (1,))) for i in R]
    Tdv = [_bf(Tm[i] * dv1[i]) for i in R]
    A_edge = [_dg(Tdv[i], Tb[i], ((0,), (0,))) * Ae[i] for i in R]
    yT = [_dg(We1_ref[...], eT[i], ((0,), (0,))) for i in R]
    yT = [_dg(_bf(yT[i]), Deb[i], ((1,), (1,))) for i in R]
    yT = [_dg(_bf(yT[i]), _bf(A_edge[i]), ((1,), (1,))) for i in R]
    e1T = [jax.nn.relu(_dg(_bf(yT[i]), Deb[i], ((1,), (1,)))) for i in R]

    de2T = [_dg(pe2_ref[...], e1T[i], ((1,), (0,))) for i in R]
    n2 = node_prop(n1, de2T, Wn2_ref[...])

    v = [jax.nn.relu(_dot(n2[i], Wv1_ref[...]) + bv1_ref[...][None, :]) for i in R]
    vm = [jnp.mean(v[i], axis=0, keepdims=True) for i in R]
    for i in R:
        out_ref[i] = (jnp.sum(vm[i] * Wv2_ref[...], axis=1, keepdims=True)
                      + bv2_ref[...][None, :])


def kernel(node, edge, node_adj, edge_adj, D_v, D_e, T,
           Wn1, We1, pe1, pv1, Wn2, We2, pe2, pv2,
           Wv1, bv1, Wv2, bv2):
    edgeT = jnp.swapaxes(edge, 1, 2)
    pe1r = pe1.reshape(1, EDGE_IN)
    pv1r = pv1.reshape(1, NODE_IN)
    pe2r = pe2.reshape(1, EDGE_OUT)
    pv2r = pv2.reshape(1, NODE_OUT)
    Wv2r = Wv2.reshape(1, NODE_OUT)
    batch = lambda *dims: pl.BlockSpec((BB,) + dims, lambda b: (b, 0, 0))
    full = lambda arr: pl.BlockSpec(arr.shape, lambda b: (0,) * arr.ndim)
    grid_spec = pl.GridSpec(
        grid=(B // BB,),
        in_specs=[
            batch(N, NODE_IN),
            batch(EDGE_IN, E),
            batch(N, N),
            batch(E, E),
            batch(N, N),
            batch(E, E),
            batch(N, E),
            full(Wn1), full(We1), full(pe1r), full(pv1r),
            full(Wn2), full(pe2r), full(pv2r),
            full(Wv1), full(bv1), full(Wv2r), full(bv2),
        ],
        out_specs=pl.BlockSpec((BB, 1, 1), lambda b: (b, 0, 0)),
    )
    out = pl.pallas_call(
        _kernel,
        grid_spec=grid_spec,
        out_shape=jax.ShapeDtypeStruct((B, 1, 1), jnp.float32),
        compiler_params=pltpu.CompilerParams(
            dimension_semantics=("arbitrary",),
        ),
    )(node, edgeT, node_adj, edge_adj, D_v, D_e, T,
      Wn1, We1, pe1r, pv1r, Wn2, pe2r, pv2r,
      Wv1, bv1, Wv2r, bv2)
    return out.reshape(B, 1)
